# Initial kernel scaffold; baseline (speedup 1.0000x reference)
import jax
import jax.numpy as jnp
from jax.experimental import pallas as pl


def kernel(x, W1, W2):
    raise NotImplementedError("write your pallas kernel here")


# speedup vs baseline: 8.2404x; 8.2404x over previous
import functools

import jax
import jax.numpy as jnp
from jax.experimental import pallas as pl

_K = 20
_INF = 3.0e38


def _dot(a, b, dims):
    return jax.lax.dot_general(a, b, (dims, ((), ())),
                               preferred_element_type=jnp.float32)


def _proj_kernel(x_ref, w1_ref, w2_ref, tt_ref, t_ref, at_ref, bs_ref):
    xb = x_ref[0]
    w1 = w1_ref[0]
    w2 = w2_ref[0]
    sc = w1.shape[0]
    tt = _dot(w1, xb, ((1,), (0,)))
    t = _dot(xb, w1, ((0,), (1,)))
    tt_ref[0] = tt
    t_ref[0] = t
    w2n = w2[:, :sc]
    w2s = w2[:, sc:]
    at_ref[0] = _dot(w2n, tt, ((1,), (0,)))
    bs_ref[0] = _dot(tt, w2s, ((0,), (1,)))


def _attn_kernel(tt_ref, t_ref, at_ref, bs_ref, out_ref, *, rt, k):
    r = pl.program_id(1)
    tt = tt_ref[0]
    sc = tt.shape[0]
    sq_all = jnp.sum(tt * tt, axis=0, keepdims=True)
    rows_t = t_ref[0, pl.ds(r * rt, rt), :]
    inner = _dot(rows_t, tt, ((1,), (0,)))
    score = sq_all - 2.0 * inner

    def body(_, prev):
        masked = jnp.where(score > prev, score, _INF)
        return jnp.min(masked, axis=1, keepdims=True)

    tau = jax.lax.fori_loop(0, k, body,
                            jnp.full((rt, 1), -_INF, jnp.float32))
    mask = score <= tau

    bs_rows = bs_ref[0, pl.ds(r * rt, rt), :]
    cols = []
    for o in range(sc):
        e = at_ref[0, o:o + 1, :] + bs_rows[:, o:o + 1]
        e = jnp.where(e >= 0, e, 0.2 * e)
        w = jnp.where(mask, jnp.exp(e), 0.0)
        denom = jnp.sum(w, axis=1, keepdims=True)
        numer = jnp.sum(w * tt[o:o + 1, :], axis=1, keepdims=True)
        cols.append(numer / denom)
    out_t = jnp.concatenate(cols, axis=1)
    ii = jax.lax.broadcasted_iota(jnp.int32, (sc, sc), 0)
    jj = jax.lax.broadcasted_iota(jnp.int32, (sc, sc), 1)
    eye = jnp.where(ii == jj, 1.0, 0.0).astype(jnp.float32)
    out16 = _dot(eye, out_t, ((1,), (1,)))
    out_ref[0] = jnp.where(out16 > 0, out16, jnp.exp(out16) - 1.0)


def _run(x, w1, w2, k):
    b, cin, n = x.shape
    h, sc, _ = w1.shape
    g = b * h
    rt = min(256, n)
    nt = n // rt
    f32 = jnp.float32

    tt, t, at, bs = pl.pallas_call(
        _proj_kernel,
        grid=(g,),
        in_specs=[
            pl.BlockSpec((1, cin, n), lambda i: (i // h, 0, 0)),
            pl.BlockSpec((1, sc, cin), lambda i: (i % h, 0, 0)),
            pl.BlockSpec((1, sc, 2 * sc), lambda i: (i % h, 0, 0)),
        ],
        out_specs=[
            pl.BlockSpec((1, sc, n), lambda i: (i, 0, 0)),
            pl.BlockSpec((1, n, sc), lambda i: (i, 0, 0)),
            pl.BlockSpec((1, sc, n), lambda i: (i, 0, 0)),
            pl.BlockSpec((1, n, sc), lambda i: (i, 0, 0)),
        ],
        out_shape=[
            jax.ShapeDtypeStruct((g, sc, n), f32),
            jax.ShapeDtypeStruct((g, n, sc), f32),
            jax.ShapeDtypeStruct((g, sc, n), f32),
            jax.ShapeDtypeStruct((g, n, sc), f32),
        ],
    )(x, w1, w2)

    out = pl.pallas_call(
        functools.partial(_attn_kernel, rt=rt, k=k),
        grid=(g, nt),
        in_specs=[
            pl.BlockSpec((1, sc, n), lambda i, r: (i, 0, 0)),
            pl.BlockSpec((1, n, sc), lambda i, r: (i, 0, 0)),
            pl.BlockSpec((1, sc, n), lambda i, r: (i, 0, 0)),
            pl.BlockSpec((1, n, sc), lambda i, r: (i, 0, 0)),
        ],
        out_specs=pl.BlockSpec((1, sc, rt), lambda i, r: (i // h, i % h, r)),
        out_shape=jax.ShapeDtypeStruct((b, h * sc, n), f32),
    )(tt, t, at, bs)
    return out


@jax.jit
def kernel(x, W1, W2):
    return _run(x, W1, W2, _K)
